# baseline (speedup 1.0000x reference)
import functools

import jax
import jax.numpy as jnp
from jax import lax
from jax.experimental import pallas as pl
from jax.experimental.pallas import tpu as pltpu
from jax.experimental.pallas import tpu_sc as plsc

F = 16
EC = 1536
NC = 2
NS = 16
NW = NC * NS


@functools.lru_cache(maxsize=None)
def _make_seg_kernel(n: int, ch0: int, ch1: int, acc_rows: int):
    zrows = acc_rows // NS
    orows = -(-(n // NS) // 8) * 8
    olast = n - (NS - 1) * orows
    assert olast > 0 and olast % 8 == 0 and zrows % 8 == 0
    mesh = plsc.VectorSubcoreMesh(core_axis_name="c", subcore_axis_name="s")

    assert ch0 % 2 == 0 and ch1 % 2 == 0

    def body(x_hbm, src_hbm, dst_hbm, init_hbm, out_hbm,
             sv0, sv1, dv0, dv1, rows_v, acc_sh, gsem, is0, is1):
        c = lax.axis_index("c")
        s = lax.axis_index("s")
        pltpu.sync_copy(init_hbm,
                        acc_sh.at[pl.ds(pl.multiple_of(s * zrows, 8), zrows)])
        plsc.subcore_barrier()

        nch = jnp.where(c == 0, ch0, ch1)
        sc_base = c * (ch0 * NS * EC)
        bufs = [(sv0, dv0, is0), (sv1, dv1, is1)]

        def idx_fire(i, sv, dv, isem):
            base = sc_base + (s * nch + i) * EC
            pltpu.async_copy(src_hbm.at[pl.ds(base, EC)], sv, isem)
            pltpu.async_copy(dst_hbm.at[pl.ds(base, EC)], dv, isem)

        for b, (sv, dv, isem) in enumerate(bufs):
            idx_fire(b, sv, dv, isem)

        def pair(k, carry):
            for b, (sv, dv, isem) in enumerate(bufs):
                i = 2 * k + b
                pltpu.make_async_copy(src_hbm.at[pl.ds(0, EC)], sv,
                                      isem).wait()
                pltpu.make_async_copy(src_hbm.at[pl.ds(0, EC)], dv,
                                      isem).wait()
                pltpu.async_copy(x_hbm.at[sv], rows_v, gsem).wait()
                pltpu.sync_copy(rows_v, acc_sh.at[dv], add=True)

                @pl.when(i + 2 < nch)
                def _prefetch():
                    idx_fire(i + 2, sv, dv, isem)
            return carry

        lax.fori_loop(0, nch // 2, pair, 0)
        plsc.subcore_barrier()
        obase = pl.multiple_of(c * n + s * orows, 8)

        @pl.when(s < NS - 1)
        def _copy_full():
            pltpu.sync_copy(acc_sh.at[pl.ds(pl.multiple_of(s * orows, 8), orows)],
                            out_hbm.at[pl.ds(obase, orows)])

        @pl.when(s == NS - 1)
        def _copy_last():
            pltpu.sync_copy(acc_sh.at[pl.ds((NS - 1) * orows, olast)],
                            out_hbm.at[pl.ds(obase, olast)])

    return pl.kernel(
        body,
        out_type=jax.ShapeDtypeStruct((NC * n, F), jnp.float32),
        mesh=mesh,
        scratch_types=(
            [pltpu.VMEM((EC,), jnp.int32)] * 4
            + [pltpu.VMEM((EC, F), jnp.float32)]
            + [pltpu.VMEM_SHARED((acc_rows, F), jnp.float32)]
            + [pltpu.SemaphoreType.DMA] * 3
        ),
        compiler_params=pltpu.CompilerParams(use_tc_tiling_on_sc=False),
    )


def _fused_mlp(partials, W0, b0, W1, n, br=1000):
    mid = W0.shape[1]
    grid = n // br

    def mm_body(p0_ref, p1_ref, w0_ref, b0_ref, w1_ref, o_ref):
        agg = p0_ref[...] + p1_ref[...]
        h = jnp.dot(agg, w0_ref[...], preferred_element_type=jnp.float32)
        h = jnp.maximum(h + b0_ref[...], 0.0)
        o_ref[...] = jnp.dot(h, w1_ref[...], preferred_element_type=jnp.float32)

    return pl.pallas_call(
        mm_body,
        grid=(grid,),
        in_specs=[
            pl.BlockSpec((br, F), lambda i: (i, 0)),
            pl.BlockSpec((br, F), lambda i: (i + grid, 0)),
            pl.BlockSpec((F, mid), lambda i: (0, 0)),
            pl.BlockSpec((1, mid), lambda i: (0, 0)),
            pl.BlockSpec((mid, F), lambda i: (0, 0)),
        ],
        out_specs=pl.BlockSpec((br, F), lambda i: (i, 0)),
        out_shape=jax.ShapeDtypeStruct((n, F), jnp.float32),
    )(partials, partials, W0, b0.reshape(1, mid), W1)


def _combine(partials, b1, n, br=2000):
    grid = n // br

    def body(p0_ref, p1_ref, b_ref, o_ref):
        o_ref[...] = p0_ref[...] + p1_ref[...] + b_ref[...]

    return pl.pallas_call(
        body,
        grid=(grid,),
        in_specs=[
            pl.BlockSpec((br, F), lambda i: (i, 0)),
            pl.BlockSpec((br, F), lambda i: (i + grid, 0)),
            pl.BlockSpec((1, F), lambda i: (0, 0)),
        ],
        out_specs=pl.BlockSpec((br, F), lambda i: (i, 0)),
        out_shape=jax.ShapeDtypeStruct((n, F), jnp.float32),
    )(partials, partials, b1.reshape(1, F))


def kernel(input, edge_index, W0, b0, W1, b1):
    n, f = input.shape
    assert f == F and n % 8 == 0
    e = edge_index.shape[1]
    per = NS * EC
    tot = -(-(-(-e // per)) // 4) * 4
    ch0 = tot // 2
    spread = 96
    ch1 = tot - ch0
    e_pad = tot * per
    pad = e_pad - e
    src = jnp.concatenate([edge_index[0], jnp.zeros((pad,), jnp.int32)])
    dst = jnp.concatenate(
        [edge_index[1], n + (jnp.arange(pad, dtype=jnp.int32) % spread)])
    acc_rows = -(-(n + 1) // (NS * 8)) * NS * 8
    zrows = acc_rows // NS

    seg = _make_seg_kernel(n, ch0, ch1, acc_rows)
    zeros = jnp.zeros((zrows, F), jnp.float32)
    p0 = seg(input, src, dst, zeros)
    h2 = _fused_mlp(p0, W0, b0, W1, n)
    p1 = seg(h2, src, dst, zeros)
    return _combine(p1, b1, n)


# speedup vs baseline: 1.0648x; 1.0648x over previous
import functools

import jax
import jax.numpy as jnp
from jax import lax
from jax.experimental import pallas as pl
from jax.experimental.pallas import tpu as pltpu
from jax.experimental.pallas import tpu_sc as plsc

F = 16
EC = 1536
NC = 2
NS = 16
NW = NC * NS


@functools.lru_cache(maxsize=None)
def _make_seg_kernel(n: int, ch0: int, ch1: int, acc_rows: int):
    zrows = acc_rows // NS
    orows = -(-(n // NS) // 8) * 8
    olast = n - (NS - 1) * orows
    assert olast > 0 and olast % 8 == 0 and zrows % 8 == 0
    mesh = plsc.VectorSubcoreMesh(core_axis_name="c", subcore_axis_name="s")

    def body(x_hbm, src_hbm, dst_hbm, init_hbm, out_hbm,
             src_v, dst_v, rows_v, acc_sh, gsem):
        c = lax.axis_index("c")
        s = lax.axis_index("s")
        pltpu.sync_copy(init_hbm,
                        acc_sh.at[pl.ds(pl.multiple_of(s * zrows, 8), zrows)])
        plsc.subcore_barrier()

        nch = jnp.where(c == 0, ch0, ch1)
        sc_base = c * (ch0 * NS * EC)

        def chunk(i, carry):
            base = sc_base + (s * nch + i) * EC
            pltpu.sync_copy(src_hbm.at[pl.ds(base, EC)], src_v)
            pltpu.sync_copy(dst_hbm.at[pl.ds(base, EC)], dst_v)
            pltpu.async_copy(x_hbm.at[src_v], rows_v, gsem).wait()
            pltpu.sync_copy(rows_v, acc_sh.at[dst_v], add=True)
            return carry

        lax.fori_loop(0, nch, chunk, 0)
        plsc.subcore_barrier()
        obase = pl.multiple_of(c * n + s * orows, 8)

        @pl.when(s < NS - 1)
        def _copy_full():
            pltpu.sync_copy(acc_sh.at[pl.ds(pl.multiple_of(s * orows, 8), orows)],
                            out_hbm.at[pl.ds(obase, orows)])

        @pl.when(s == NS - 1)
        def _copy_last():
            pltpu.sync_copy(acc_sh.at[pl.ds((NS - 1) * orows, olast)],
                            out_hbm.at[pl.ds(obase, olast)])

    return pl.kernel(
        body,
        out_type=jax.ShapeDtypeStruct((NC * n, F), jnp.float32),
        mesh=mesh,
        scratch_types=[
            pltpu.VMEM((EC,), jnp.int32),
            pltpu.VMEM((EC,), jnp.int32),
            pltpu.VMEM((EC, F), jnp.float32),
            pltpu.VMEM_SHARED((acc_rows, F), jnp.float32),
            pltpu.SemaphoreType.DMA,
        ],
        compiler_params=pltpu.CompilerParams(use_tc_tiling_on_sc=False),
    )


def _fused_mlp(partials, W0, b0, W1, n, br=1000):
    mid = W0.shape[1]
    grid = n // br

    def mm_body(p0_ref, p1_ref, w0_ref, b0_ref, w1_ref, o_ref):
        agg = p0_ref[...] + p1_ref[...]
        h = jnp.dot(agg, w0_ref[...], preferred_element_type=jnp.float32)
        h = jnp.maximum(h + b0_ref[...], 0.0)
        o_ref[...] = jnp.dot(h, w1_ref[...], preferred_element_type=jnp.float32)

    return pl.pallas_call(
        mm_body,
        grid=(grid,),
        in_specs=[
            pl.BlockSpec((br, F), lambda i: (i, 0)),
            pl.BlockSpec((br, F), lambda i: (i + grid, 0)),
            pl.BlockSpec((F, mid), lambda i: (0, 0)),
            pl.BlockSpec((1, mid), lambda i: (0, 0)),
            pl.BlockSpec((mid, F), lambda i: (0, 0)),
        ],
        out_specs=pl.BlockSpec((br, F), lambda i: (i, 0)),
        out_shape=jax.ShapeDtypeStruct((n, F), jnp.float32),
    )(partials, partials, W0, b0.reshape(1, mid), W1)


def _combine(partials, b1, n, br=2000):
    grid = n // br

    def body(p0_ref, p1_ref, b_ref, o_ref):
        o_ref[...] = p0_ref[...] + p1_ref[...] + b_ref[...]

    return pl.pallas_call(
        body,
        grid=(grid,),
        in_specs=[
            pl.BlockSpec((br, F), lambda i: (i, 0)),
            pl.BlockSpec((br, F), lambda i: (i + grid, 0)),
            pl.BlockSpec((1, F), lambda i: (0, 0)),
        ],
        out_specs=pl.BlockSpec((br, F), lambda i: (i, 0)),
        out_shape=jax.ShapeDtypeStruct((n, F), jnp.float32),
    )(partials, partials, b1.reshape(1, F))


def kernel(input, edge_index, W0, b0, W1, b1):
    n, f = input.shape
    assert f == F and n % 8 == 0
    e = edge_index.shape[1]
    per = NS * EC
    tot = -(-e // per)
    ch0 = tot // 2
    spread = 96
    ch1 = tot - ch0
    e_pad = tot * per
    pad = e_pad - e
    src = jnp.concatenate([edge_index[0], jnp.zeros((pad,), jnp.int32)])
    dst = jnp.concatenate(
        [edge_index[1], n + (jnp.arange(pad, dtype=jnp.int32) % spread)])
    acc_rows = -(-(n + 1) // (NS * 8)) * NS * 8
    zrows = acc_rows // NS

    seg = _make_seg_kernel(n, ch0, ch1, acc_rows)
    zeros = jnp.zeros((zrows, F), jnp.float32)
    p0 = seg(input, src, dst, zeros)
    h2 = _fused_mlp(p0, W0, b0, W1, n)
    p1 = seg(h2, src, dst, zeros)
    return _combine(p1, b1, n)


# speedup vs baseline: 1.1488x; 1.0789x over previous
import functools

import jax
import jax.numpy as jnp
from jax import lax
from jax.experimental import pallas as pl
from jax.experimental.pallas import tpu as pltpu
from jax.experimental.pallas import tpu_sc as plsc

F = 16
EC = 1000
NC = 2
NS = 16
NW = NC * NS


@functools.lru_cache(maxsize=None)
def _make_seg_kernel(n: int, ch0: int, ch1: int, acc_rows: int):
    zrows = acc_rows // NS
    orows = -(-(n // NS) // 8) * 8
    olast = n - (NS - 1) * orows
    assert olast > 0 and olast % 8 == 0 and zrows % 8 == 0
    mesh = plsc.VectorSubcoreMesh(core_axis_name="c", subcore_axis_name="s")

    def body(x_hbm, src_hbm, dst_hbm, init_hbm, out_hbm,
             src_v, dst_v, rows_v, acc_sh, gsem):
        c = lax.axis_index("c")
        s = lax.axis_index("s")
        pltpu.sync_copy(init_hbm,
                        acc_sh.at[pl.ds(pl.multiple_of(s * zrows, 8), zrows)])
        plsc.subcore_barrier()

        nch = jnp.where(c == 0, ch0, ch1)
        sc_base = c * (ch0 * NS * EC)

        def chunk(i, carry):
            base = sc_base + (s * nch + i) * EC
            pltpu.sync_copy(src_hbm.at[pl.ds(base, EC)], src_v)
            pltpu.sync_copy(dst_hbm.at[pl.ds(base, EC)], dst_v)
            pltpu.async_copy(x_hbm.at[src_v], rows_v, gsem).wait()
            pltpu.sync_copy(rows_v, acc_sh.at[dst_v], add=True)
            return carry

        lax.fori_loop(0, nch, chunk, 0)
        plsc.subcore_barrier()
        obase = pl.multiple_of(c * n + s * orows, 8)

        @pl.when(s < NS - 1)
        def _copy_full():
            pltpu.sync_copy(acc_sh.at[pl.ds(pl.multiple_of(s * orows, 8), orows)],
                            out_hbm.at[pl.ds(obase, orows)])

        @pl.when(s == NS - 1)
        def _copy_last():
            pltpu.sync_copy(acc_sh.at[pl.ds((NS - 1) * orows, olast)],
                            out_hbm.at[pl.ds(obase, olast)])

    return pl.kernel(
        body,
        out_type=jax.ShapeDtypeStruct((NC * n, F), jnp.float32),
        mesh=mesh,
        scratch_types=[
            pltpu.VMEM((EC,), jnp.int32),
            pltpu.VMEM((EC,), jnp.int32),
            pltpu.VMEM((EC, F), jnp.float32),
            pltpu.VMEM_SHARED((acc_rows, F), jnp.float32),
            pltpu.SemaphoreType.DMA,
        ],
        compiler_params=pltpu.CompilerParams(use_tc_tiling_on_sc=False),
    )


def _fused_mlp(partials, W0, b0, W1, n, br=1000):
    mid = W0.shape[1]
    grid = n // br

    def mm_body(p0_ref, p1_ref, w0_ref, b0_ref, w1_ref, o_ref):
        agg = p0_ref[...] + p1_ref[...]
        h = jnp.dot(agg, w0_ref[...], preferred_element_type=jnp.float32)
        h = jnp.maximum(h + b0_ref[...], 0.0)
        o_ref[...] = jnp.dot(h, w1_ref[...], preferred_element_type=jnp.float32)

    return pl.pallas_call(
        mm_body,
        grid=(grid,),
        in_specs=[
            pl.BlockSpec((br, F), lambda i: (i, 0)),
            pl.BlockSpec((br, F), lambda i: (i + grid, 0)),
            pl.BlockSpec((F, mid), lambda i: (0, 0)),
            pl.BlockSpec((1, mid), lambda i: (0, 0)),
            pl.BlockSpec((mid, F), lambda i: (0, 0)),
        ],
        out_specs=pl.BlockSpec((br, F), lambda i: (i, 0)),
        out_shape=jax.ShapeDtypeStruct((n, F), jnp.float32),
    )(partials, partials, W0, b0.reshape(1, mid), W1)


def _combine(partials, b1, n, br=2000):
    grid = n // br

    def body(p0_ref, p1_ref, b_ref, o_ref):
        o_ref[...] = p0_ref[...] + p1_ref[...] + b_ref[...]

    return pl.pallas_call(
        body,
        grid=(grid,),
        in_specs=[
            pl.BlockSpec((br, F), lambda i: (i, 0)),
            pl.BlockSpec((br, F), lambda i: (i + grid, 0)),
            pl.BlockSpec((1, F), lambda i: (0, 0)),
        ],
        out_specs=pl.BlockSpec((br, F), lambda i: (i, 0)),
        out_shape=jax.ShapeDtypeStruct((n, F), jnp.float32),
    )(partials, partials, b1.reshape(1, F))


def kernel(input, edge_index, W0, b0, W1, b1):
    n, f = input.shape
    assert f == F and n % 8 == 0
    e = edge_index.shape[1]
    per = NS * EC
    tot = -(-e // per)
    ch0 = tot // 2
    spread = 96
    ch1 = tot - ch0
    e_pad = tot * per
    pad = e_pad - e
    if pad:
        src = jnp.concatenate([edge_index[0], jnp.zeros((pad,), jnp.int32)])
        dst = jnp.concatenate(
            [edge_index[1], n + (jnp.arange(pad, dtype=jnp.int32) % spread)])
    else:
        src = edge_index[0]
        dst = edge_index[1]
    acc_rows = -(-(n + 1) // (NS * 8)) * NS * 8
    zrows = acc_rows // NS

    seg = _make_seg_kernel(n, ch0, ch1, acc_rows)
    zeros = jnp.zeros((zrows, F), jnp.float32)
    p0 = seg(input, src, dst, zeros)
    h2 = _fused_mlp(p0, W0, b0, W1, n)
    p1 = seg(h2, src, dst, zeros)
    return _combine(p1, b1, n)
